# Initial kernel scaffold; baseline (speedup 1.0000x reference)
import jax
import jax.numpy as jnp
from jax.experimental import pallas as pl


def kernel(x, edge_index, W):
    raise NotImplementedError("write your pallas kernel here")


# speedup vs baseline: 3.9276x; 3.9276x over previous
import jax
import jax.numpy as jnp
from jax import lax
from jax.experimental import pallas as pl
from jax.experimental.pallas import tpu as pltpu
from jax.experimental.pallas import tpu_sc as plsc

N_NODES = 10000
N_PAD = 10240
N_EDGES = 320000
D = 128
DH = D // 2

NC = 2
NS = 16
E_PER_S = N_EDGES // NS
CHUNK = 80
N_CHUNKS = E_PER_S // CHUNK
ROWS_PER_TILE = N_PAD // NS
CNT_W = 16


def _sc_kernel(xlo_hbm, xhi_hbm, src_hbm, dst_hbm, acc_out, cnt_out,
               src_idx, dst_idx, rows, ones_v, zacc, zcnt,
               acc_sh, cnt_sh, sem):
    cid = lax.axis_index("c")
    sid = lax.axis_index("s")

    zero16 = jnp.zeros((16,), jnp.float32)
    one16 = jnp.ones((16,), jnp.float32)

    def init_zacc(i, carry):
        for c8 in range(DH // 16):
            zacc[i, pl.ds(c8 * 16, 16)] = zero16
        return carry

    lax.fori_loop(0, 128, init_zacc, 0)

    def init_zcnt(i, carry):
        zcnt[i, :] = zero16
        return carry

    lax.fori_loop(0, ROWS_PER_TILE, init_zcnt, 0)

    def init_ones(i, carry):
        ones_v[i, :] = one16
        return carry

    lax.fori_loop(0, CHUNK, init_ones, 0)

    for b in range(ROWS_PER_TILE // 128):
        pltpu.sync_copy(zacc, acc_sh.at[pl.ds(sid * ROWS_PER_TILE + b * 128, 128)])
    pltpu.sync_copy(zcnt, cnt_sh.at[pl.ds(sid * ROWS_PER_TILE, ROWS_PER_TILE)])

    plsc.subcore_barrier()

    ebase = sid * E_PER_S

    def chunk_body(i, carry):
        base = ebase + i * CHUNK
        pltpu.sync_copy(src_hbm.at[pl.ds(base, CHUNK)], src_idx)
        pltpu.sync_copy(dst_hbm.at[pl.ds(base, CHUNK)], dst_idx)

        @pl.when(cid == 0)
        def _():
            pltpu.async_copy(xlo_hbm.at[src_idx], rows, sem).wait()

        @pl.when(cid == 1)
        def _():
            pltpu.async_copy(xhi_hbm.at[src_idx], rows, sem).wait()

        pltpu.sync_copy(rows, acc_sh.at[dst_idx], add=True)

        @pl.when(cid == 0)
        def _():
            pltpu.sync_copy(ones_v, cnt_sh.at[dst_idx], add=True)

        return carry

    lax.fori_loop(0, N_CHUNKS, chunk_body, 0)

    plsc.subcore_barrier()

    rbase = sid * ROWS_PER_TILE
    pltpu.sync_copy(acc_sh.at[pl.ds(rbase, ROWS_PER_TILE)],
                    acc_out.at[cid, pl.ds(rbase, ROWS_PER_TILE)])

    @pl.when(cid == 0)
    def _():
        pltpu.sync_copy(cnt_sh.at[pl.ds(rbase, ROWS_PER_TILE)],
                        cnt_out.at[pl.ds(rbase, ROWS_PER_TILE)])


@jax.jit
def _sc_aggregate(xlo, xhi, src, dst):
    mesh = plsc.VectorSubcoreMesh(core_axis_name="c", subcore_axis_name="s")
    f = pl.kernel(
        _sc_kernel,
        out_type=[
            jax.ShapeDtypeStruct((NC, N_PAD, DH), jnp.float32),
            jax.ShapeDtypeStruct((N_PAD, CNT_W), jnp.float32),
        ],
        mesh=mesh,
        scratch_types=[
            pltpu.VMEM((CHUNK,), jnp.int32),
            pltpu.VMEM((CHUNK,), jnp.int32),
            pltpu.VMEM((CHUNK, DH), jnp.float32),
            pltpu.VMEM((CHUNK, CNT_W), jnp.float32),
            pltpu.VMEM((128, DH), jnp.float32),
            pltpu.VMEM((ROWS_PER_TILE, CNT_W), jnp.float32),
            pltpu.VMEM_SHARED((N_PAD, DH), jnp.float32),
            pltpu.VMEM_SHARED((N_PAD, CNT_W), jnp.float32),
            pltpu.SemaphoreType.DMA,
        ],
        compiler_params=pltpu.CompilerParams(use_tc_tiling_on_sc=False),
    )
    return f(xlo, xhi, src, dst)


def _tc_finish_body(acc_ref, cnt_ref, w_ref, o_ref):
    s = jnp.concatenate([acc_ref[0], acc_ref[1]], axis=1)
    c = cnt_ref[:, 0]
    z = lax.dot_general(s, w_ref[...], (((1,), (1,)), ((), ())),
                        preferred_element_type=jnp.float32)
    o_ref[...] = z / jnp.maximum(c, 1.0)[:, None]


@jax.jit
def _tc_finish(acc, cnt, W):
    blk = 1024
    return pl.pallas_call(
        _tc_finish_body,
        grid=(N_PAD // blk,),
        in_specs=[
            pl.BlockSpec((NC, blk, DH), lambda i: (0, i, 0)),
            pl.BlockSpec((blk, CNT_W), lambda i: (i, 0)),
            pl.BlockSpec((D, D), lambda i: (0, 0)),
        ],
        out_specs=pl.BlockSpec((blk, D), lambda i: (i, 0)),
        out_shape=jax.ShapeDtypeStruct((N_PAD, D), jnp.float32),
    )(acc, cnt, W)


def kernel(x, edge_index, W):
    src = edge_index[0]
    dst = edge_index[1]
    xlo = x[:, :DH]
    xhi = x[:, DH:]
    acc, cnt = _sc_aggregate(xlo, xhi, src, dst)
    return _tc_finish(acc, cnt, W)[:N_NODES]


# speedup vs baseline: 7.9442x; 2.0226x over previous
import jax
import jax.numpy as jnp
from jax import lax
from jax.experimental import pallas as pl
from jax.experimental.pallas import tpu as pltpu
from jax.experimental.pallas import tpu_sc as plsc

N_NODES = 10000
N_PAD = 10240
N_EDGES = 320000
D = 128
DH = D // 2

NC = 2
NS = 16
E_PER_S = N_EDGES // NS
CHUNK = 80
N_CHUNKS = E_PER_S // CHUNK
ROWS_PER_TILE = N_PAD // NS
CNT_W = 16


def _sc_kernel(xlo_hbm, xhi_hbm, src_hbm, dst_hbm, acc_out, cnt_out,
               src_all, dst_all, rows0, rows1, ones_v, zacc, zcnt,
               acc_sh, cnt_sh, sem0, sem1):
    cid = lax.axis_index("c")
    sid = lax.axis_index("s")

    zero16 = jnp.zeros((16,), jnp.float32)
    one16 = jnp.ones((16,), jnp.float32)

    def init_zacc(i, carry):
        for c8 in range(DH // 16):
            zacc[i, pl.ds(c8 * 16, 16)] = zero16
        return carry

    lax.fori_loop(0, 128, init_zacc, 0)

    def init_zcnt(i, carry):
        zcnt[i, :] = zero16
        return carry

    lax.fori_loop(0, ROWS_PER_TILE, init_zcnt, 0)

    def init_ones(i, carry):
        ones_v[i, :] = one16
        return carry

    lax.fori_loop(0, CHUNK, init_ones, 0)

    for b in range(ROWS_PER_TILE // 128):
        pltpu.sync_copy(zacc, acc_sh.at[pl.ds(sid * ROWS_PER_TILE + b * 128, 128)])
    pltpu.sync_copy(zcnt, cnt_sh.at[pl.ds(sid * ROWS_PER_TILE, ROWS_PER_TILE)])

    plsc.subcore_barrier()

    ebase = sid * E_PER_S

    pltpu.sync_copy(src_hbm.at[pl.ds(ebase, E_PER_S)], src_all)
    pltpu.sync_copy(dst_hbm.at[pl.ds(ebase, E_PER_S)], dst_all)

    def gather(i, buf, sem):
        idx = src_all.at[pl.ds(i * CHUNK, CHUNK)]

        @pl.when(cid == 0)
        def _():
            pltpu.async_copy(xlo_hbm.at[idx], buf, sem)

        @pl.when(cid == 1)
        def _():
            pltpu.async_copy(xhi_hbm.at[idx], buf, sem)

    def gwait(buf, sem):
        pltpu.make_async_copy(xlo_hbm.at[pl.ds(0, CHUNK)], buf, sem).wait()

    gather(0, rows0, sem0)

    def chunk_body(i, carry):
        buf, sem = rows0, sem0
        nbuf, nsem = rows1, sem1
        gwait(buf, sem)

        @pl.when(i + 1 < N_CHUNKS)
        def _():
            gather(i + 1, nbuf, nsem)

        didx = dst_all.at[pl.ds(i * CHUNK, CHUNK)]
        pltpu.sync_copy(buf, acc_sh.at[didx], add=True)

        @pl.when(cid == 0)
        def _():
            pltpu.sync_copy(ones_v, cnt_sh.at[didx], add=True)

        return carry

    def two_chunks(j, carry):
        chunk_body(2 * j, carry)
        i = 2 * j + 1
        gwait(rows1, sem1)

        @pl.when(i + 1 < N_CHUNKS)
        def _():
            gather(i + 1, rows0, sem0)

        didx = dst_all.at[pl.ds(i * CHUNK, CHUNK)]
        pltpu.sync_copy(rows1, acc_sh.at[didx], add=True)

        @pl.when(cid == 0)
        def _():
            pltpu.sync_copy(ones_v, cnt_sh.at[didx], add=True)

        return carry

    lax.fori_loop(0, N_CHUNKS // 2, two_chunks, 0)

    plsc.subcore_barrier()

    rbase = sid * ROWS_PER_TILE
    pltpu.sync_copy(acc_sh.at[pl.ds(rbase, ROWS_PER_TILE)],
                    acc_out.at[cid, pl.ds(rbase, ROWS_PER_TILE)])

    @pl.when(cid == 0)
    def _():
        pltpu.sync_copy(cnt_sh.at[pl.ds(rbase, ROWS_PER_TILE)],
                        cnt_out.at[pl.ds(rbase, ROWS_PER_TILE)])


@jax.jit
def _sc_aggregate(xlo, xhi, src, dst):
    mesh = plsc.VectorSubcoreMesh(core_axis_name="c", subcore_axis_name="s")
    f = pl.kernel(
        _sc_kernel,
        out_type=[
            jax.ShapeDtypeStruct((NC, N_PAD, DH), jnp.float32),
            jax.ShapeDtypeStruct((N_PAD, CNT_W), jnp.float32),
        ],
        mesh=mesh,
        scratch_types=[
            pltpu.VMEM((E_PER_S,), jnp.int32),
            pltpu.VMEM((E_PER_S,), jnp.int32),
            pltpu.VMEM((CHUNK, DH), jnp.float32),
            pltpu.VMEM((CHUNK, DH), jnp.float32),
            pltpu.VMEM((CHUNK, CNT_W), jnp.float32),
            pltpu.VMEM((128, DH), jnp.float32),
            pltpu.VMEM((ROWS_PER_TILE, CNT_W), jnp.float32),
            pltpu.VMEM_SHARED((N_PAD, DH), jnp.float32),
            pltpu.VMEM_SHARED((N_PAD, CNT_W), jnp.float32),
            pltpu.SemaphoreType.DMA,
            pltpu.SemaphoreType.DMA,
        ],
        compiler_params=pltpu.CompilerParams(use_tc_tiling_on_sc=False),
    )
    return f(xlo, xhi, src, dst)


def _tc_finish_body(acc_ref, cnt_ref, w_ref, o_ref):
    s = jnp.concatenate([acc_ref[0], acc_ref[1]], axis=1)
    c = cnt_ref[:, 0]
    z = lax.dot_general(s, w_ref[...], (((1,), (1,)), ((), ())),
                        preferred_element_type=jnp.float32)
    o_ref[...] = z / jnp.maximum(c, 1.0)[:, None]


@jax.jit
def _tc_finish(acc, cnt, W):
    blk = 1024
    return pl.pallas_call(
        _tc_finish_body,
        grid=(N_PAD // blk,),
        in_specs=[
            pl.BlockSpec((NC, blk, DH), lambda i: (0, i, 0)),
            pl.BlockSpec((blk, CNT_W), lambda i: (i, 0)),
            pl.BlockSpec((D, D), lambda i: (0, 0)),
        ],
        out_specs=pl.BlockSpec((blk, D), lambda i: (i, 0)),
        out_shape=jax.ShapeDtypeStruct((N_PAD, D), jnp.float32),
    )(acc, cnt, W)


def kernel(x, edge_index, W):
    src = edge_index[0]
    dst = edge_index[1]
    xlo = x[:, :DH]
    xhi = x[:, DH:]
    acc, cnt = _sc_aggregate(xlo, xhi, src, dst)
    return _tc_finish(acc, cnt, W)[:N_NODES]
